# baseline (speedup 1.0000x reference)
import functools

import numpy as np
import jax
import jax.numpy as jnp
from jax import lax
from jax.experimental import pallas as pl
from jax.experimental.pallas import tpu as pltpu
from jax.experimental.pallas import tpu_sc as plsc

D_MODEL = 768
MAX_SEQ_LEN = 4096
BATCH = 4
SEQ_LEN = 4096
N_ROWS = BATCH * SEQ_LEN

NUM_CORES = 2
NUM_SUBCORES = 16
LANES = 16
NUM_WORKERS = NUM_CORES * NUM_SUBCORES
ROWS_PER_WORKER = N_ROWS // NUM_WORKERS
CHUNK = 32
NUM_CHUNKS = ROWS_PER_WORKER // CHUNK

SCALE = float(np.sqrt(np.float32(D_MODEL)))


def _sinusoidal_pe_np(max_seq_len, d_model):
    position = np.arange(0, max_seq_len, dtype=np.float32)[:, None]
    div_term = np.exp(
        np.arange(0, d_model, 2).astype(np.float32) * (-np.log(10000.0) / d_model)
    )
    pe = np.zeros((max_seq_len, d_model), dtype=np.float32)
    pe[:, 0::2] = np.sin(position * div_term)
    pe[:, 1::2] = np.cos(position * div_term)
    return pe


_PE = _sinusoidal_pe_np(MAX_SEQ_LEN, D_MODEL)


_MESH = plsc.VectorSubcoreMesh(core_axis_name="c", subcore_axis_name="s")


@functools.partial(
    pl.kernel,
    mesh=_MESH,
    out_type=jax.ShapeDtypeStruct((N_ROWS, D_MODEL), jnp.float32),
    scratch_types=[
        pltpu.VMEM((ROWS_PER_WORKER,), jnp.int32),
        pltpu.VMEM((CHUNK, D_MODEL), jnp.float32),
        pltpu.VMEM((CHUNK, D_MODEL), jnp.float32),
        pltpu.SemaphoreType.DMA,
    ],
)
def _embed_sc(x_hbm, table_hbm, pe_hbm, out_hbm, idx_v, rows_v, acc_v, sem):
    wid = lax.axis_index("s") * NUM_CORES + lax.axis_index("c")
    base = wid * ROWS_PER_WORKER
    pos_base = base % SEQ_LEN

    pltpu.sync_copy(x_hbm.at[pl.ds(base, ROWS_PER_WORKER)], idx_v)

    def chunk_body(g, carry):
        r0 = g * CHUNK
        pltpu.async_copy(
            table_hbm.at[idx_v.at[pl.ds(r0, CHUNK)]], rows_v, sem
        ).wait()
        pltpu.sync_copy(pe_hbm.at[pl.ds(pos_base + r0, CHUNK)], acc_v)

        def row_body(r, c):
            for j in range(D_MODEL // LANES):
                v = rows_v[r, pl.ds(j * LANES, LANES)]
                plsc.addupdate(acc_v.at[r, pl.ds(j * LANES, LANES)], v * SCALE)
            return c

        lax.fori_loop(0, CHUNK, row_body, 0)

        pltpu.sync_copy(acc_v, out_hbm.at[pl.ds(base + r0, CHUNK)])
        return carry

    lax.fori_loop(0, NUM_CHUNKS, chunk_body, 0)


def kernel(x, table):
    xf = x.reshape(N_ROWS).astype(jnp.int32)
    pe = jnp.asarray(_PE)
    out = _embed_sc(xf, table, pe)
    return out.reshape(BATCH, SEQ_LEN, D_MODEL)
